# Initial kernel scaffold; baseline (speedup 1.0000x reference)
import jax
import jax.numpy as jnp
from jax.experimental import pallas as pl


def kernel(input_ids, tok_table, pos_table, gamma, beta):
    raise NotImplementedError("write your pallas kernel here")


# speedup vs baseline: 1.0799x; 1.0799x over previous
import functools

import jax
import jax.numpy as jnp
from jax import lax
from jax.experimental import pallas as pl
from jax.experimental.pallas import tpu as pltpu
from jax.experimental.pallas import tpu_sc as plsc

EPS = 1e-12


def _sc_gather(tok_table, ids, n_rows, d):
    info = plsc.get_sparse_core_info()
    nw = info.num_cores * info.num_subcores
    rows_per_w = n_rows // nw
    chunk = 128

    mesh = plsc.VectorSubcoreMesh(core_axis_name="c", subcore_axis_name="s")

    @functools.partial(
        pl.kernel,
        mesh=mesh,
        out_type=jax.ShapeDtypeStruct((n_rows, d), jnp.float32),
        scratch_types=[
            pltpu.VMEM((chunk,), jnp.int32),
            pltpu.VMEM((chunk, d), jnp.float32),
            pltpu.SemaphoreType.DMA,
        ],
    )
    def gather_kernel(table_hbm, idx_hbm, out_hbm, idx_v, rows_v, sem):
        wid = lax.axis_index("s") * info.num_cores + lax.axis_index("c")
        base = wid * rows_per_w

        @pl.loop(0, rows_per_w, step=chunk)
        def _(c):
            pltpu.sync_copy(idx_hbm.at[pl.ds(base + c, chunk)], idx_v)
            pltpu.async_copy(table_hbm.at[idx_v], rows_v, sem).wait()
            pltpu.sync_copy(rows_v, out_hbm.at[pl.ds(base + c, chunk)])

    return gather_kernel(tok_table, ids)


def _tc_add_ln(gathered, pos_table, gamma2, beta2, n_rows, t_len, d, blk):
    pos_blocks = t_len // blk

    def body(g_ref, p_ref, gm_ref, bt_ref, o_ref):
        i = pl.program_id(0)
        h = g_ref[...] + p_ref[pl.ds((i % pos_blocks) * blk, blk), :]
        mu = jnp.mean(h, axis=1, keepdims=True)
        hc = h - mu
        var = jnp.mean(hc * hc, axis=1, keepdims=True)
        o_ref[...] = hc * lax.rsqrt(var + EPS) * gm_ref[...] + bt_ref[...]

    return pl.pallas_call(
        body,
        grid=(n_rows // blk,),
        in_specs=[
            pl.BlockSpec((blk, d), lambda i: (i, 0)),
            pl.BlockSpec((t_len, d), lambda i: (0, 0)),
            pl.BlockSpec((1, d), lambda i: (0, 0)),
            pl.BlockSpec((1, d), lambda i: (0, 0)),
        ],
        out_specs=pl.BlockSpec((blk, d), lambda i: (i, 0)),
        out_shape=jax.ShapeDtypeStruct((n_rows, d), jnp.float32),
    )(gathered, pos_table, gamma2, beta2)


def kernel(input_ids, tok_table, pos_table, gamma, beta):
    b, t = input_ids.shape
    v, d = tok_table.shape
    n_rows = b * t

    ids = input_ids.reshape(-1).astype(jnp.int32)
    gathered = _sc_gather(tok_table, ids, n_rows, d)
    out = _tc_add_ln(
        gathered,
        pos_table,
        gamma.reshape(1, d),
        beta.reshape(1, d),
        n_rows,
        t,
        d,
        blk=256,
    )
    return out.reshape(b, t, d)
